# baseline (speedup 1.0000x reference)
import functools

import jax
import jax.numpy as jnp
from jax import lax
from jax.experimental import pallas as pl
from jax.experimental.pallas import tpu as pltpu
from jax.experimental.pallas import tpu_sc as plsc

N = 256
E = 8192
D = 128
NC = 2
NS = 16
NW = NC * NS
EPW = E // NW
CH = 64
NCH = EPW // CH
L = 16


def _elu1(x):
    return jnp.where(x > 0, x + 1.0, jnp.exp(x))


def _ln(x, g, b):
    mu = x.mean(-1, keepdims=True)
    var = ((x - mu) ** 2).mean(-1, keepdims=True)
    return g * (x - mu) / jnp.sqrt(var + 1e-5) + b


def _tc_prep(h, e, emb_h, A1, Ab1, B1, Bb1, C1, Cb1, D1, Db1, E1, Eb1,
             ah_o, dbh_o, eh_o, ce1_o):
    hh = h[...]
    ah_o[...] = hh @ A1[...] + Ab1[...]
    dbh_o[:, :D] = hh @ D1[...] + Db1[...]
    dbh_o[:, D:] = hh @ B1[...] + Bb1[...]
    eh_o[...] = hh @ E1[...] + Eb1[...]
    ce1_o[...] = e[...] @ C1[...] + Cb1[...]


def _tc_prep_b(e, emb_h, C2, Cb2, Ms, Md, W2l, b2l, b4, bng, bnb,
               ce2_o, shat_o, ldw_o):
    ce2_o[...] = e[...] @ C2[...] + Cb2[...]
    emb = emb_h[...]
    Ls = (emb @ Ms[...]) @ W2l[...]
    LdW = (emb @ Md[...]) @ W2l[...]
    gam = bng[...] / jnp.sqrt(1.0 + 1e-5)
    shat_o[...] = gam * (Ls + b2l[...] + b4[...]) + bnb[...]
    ldw_o[...] = LdW


def _tc_mid(h, ah1, ndp, g1, b1_, A2, Ab2, B2, Bb2, D2, Db2,
            E2, Eb2,
            h1_o, ah_o, dbh_o, eh_o):
    num = ndp[0, 0] + ndp[1, 0]
    den = ndp[0, 1] + ndp[1, 1]
    h_new = ah1[...] + num / (den + 1e-6)
    hb = g1[...] * h_new / jnp.sqrt(1.0 + 1e-5) + b1_[...]
    h1 = h[...] + jnp.maximum(hb, 0.0)
    h1_o[...] = h1
    ah_o[...] = h1 @ A2[...] + Ab2[...]
    dbh_o[:, :D] = h1 @ D2[...] + Db2[...]
    dbh_o[:, D:] = h1 @ B2[...] + Bb2[...]
    eh_o[...] = h1 @ E2[...] + Eb2[...]


def _tc_tail_pre(shat, ldw, srcc, dstc, bng, se_o):
    gam = bng[...] / jnp.sqrt(1.0 + 1e-5)
    col = lax.broadcasted_iota(jnp.int32, (E, N), 1)
    oh_src = (srcc[...] == col).astype(jnp.float32)
    oh_dst = (dstc[...] == col).astype(jnp.float32)
    se_o[...] = (jnp.dot(oh_src, shat[...], preferred_element_type=jnp.float32)
                 + jnp.dot(oh_dst, gam * ldw[...],
                           preferred_element_type=jnp.float32))


def _tc_tail(h1, ah2, ndp, g2, b2_, emb_h,
             Wq, bq, Wk, bk, Wv, bv, Wm, bm, ln1g, ln1b,
             W1, b1e, W2e, b2e, ln2g, ln2b, W4, bng,
             se, dstc,
             lre_o):
    num = ndp[0, 0] + ndp[1, 0]
    den = ndp[0, 1] + ndp[1, 1]
    h_new = ah2[...] + num / (den + 1e-6)
    hb = g2[...] * h_new / jnp.sqrt(1.0 + 1e-5) + b2_[...]
    h2 = h1[...] + jnp.maximum(hb, 0.0)
    g = h2.mean(0, keepdims=True)

    emb = emb_h[...]
    Q = _elu1(emb @ Wq[...] + bq[...])
    krow = _elu1(g @ Wk[...] + bk[...])
    vrow = g @ Wv[...] + bv[...]
    s = (Q * krow).sum(-1, keepdims=True)
    ns = jnp.float32(N) * s
    msg = (ns / (ns + 1e-6)) * vrow
    msg = _ln(msg @ Wm[...] + bm[...], ln1g[...], ln1b[...])
    y = jnp.concatenate([emb, msg], axis=-1)
    y = jnp.maximum(y @ W1[...] + b1e[...], 0.0) @ W2e[...] + b2e[...]
    qfea = emb + _ln(y, ln2g[...], ln2b[...])

    Q2 = _elu1(qfea @ Wq[...] + bq[...])
    K2 = _elu1(qfea @ Wk[...] + bk[...])
    V2 = qfea @ Wv[...] + bv[...]
    KV = jnp.float32(N) * lax.dot_general(
        K2, V2, (((0,), (0,)), ((), ())), preferred_element_type=jnp.float32)
    ksum = jnp.float32(N) * K2.sum(0, keepdims=True)
    Z = (Q2 * ksum).sum(-1, keepdims=True)
    msg2 = (Q2 @ KV) / (Z + 1e-6)
    msg2 = _ln(msg2 @ Wm[...] + bm[...], ln1g[...], ln1b[...])
    y2 = jnp.concatenate([qfea, msg2], axis=-1)
    y2 = jnp.maximum(y2 @ W1[...] + b1e[...], 0.0) @ W2e[...] + b2e[...]
    arm = qfea + _ln(y2, ln2g[...], ln2b[...])

    gam = bng[...] / jnp.sqrt(1.0 + 1e-5)
    G = gam * (arm @ W4[...])

    col = lax.broadcasted_iota(jnp.int32, (E, N), 1)
    oh_dst = (dstc[...] == col).astype(jnp.float32)
    lre_o[...] = jnp.maximum(
        se[...] + jnp.dot(oh_dst, G, preferred_element_type=jnp.float32),
        0.0)


_sc_mesh = plsc.VectorSubcoreMesh(core_axis_name="c", subcore_axis_name="s")


@functools.partial(
    pl.kernel, mesh=_sc_mesh,
    out_type=jax.ShapeDtypeStruct((NC, 2, N, D), jnp.float32),
    scratch_types=[
        pltpu.VMEM((EPW,), jnp.int32),
        pltpu.VMEM((NCH, CH), jnp.int32),
        pltpu.VMEM((CH, 2 * D), jnp.float32),
        pltpu.VMEM((CH, 2 * D), jnp.float32),
        pltpu.VMEM((CH, D), jnp.float32),
        pltpu.VMEM((CH, D), jnp.float32),
        pltpu.VMEM((CH, D), jnp.float32),
        pltpu.VMEM((CH, D), jnp.float32),
        pltpu.VMEM((CH, D), jnp.float32),
        pltpu.VMEM((CH, D), jnp.float32),
        pltpu.VMEM((CH, D), jnp.float32),
        pltpu.VMEM((CH, D), jnp.float32),
        pltpu.VMEM_SHARED((N, D), jnp.float32),
        pltpu.VMEM_SHARED((N, D), jnp.float32),
        pltpu.SemaphoreType.DMA,
        pltpu.SemaphoreType.DMA,
        pltpu.SemaphoreType.DMA,
    ])
def _sc_edge_pass(dbh, eh, ce, src_h, dst_h, zeros_h,
                  nd_o,
                  src_v, dst_v, db0, db1, e0, e1, c0, c1,
                  sb0, sb1, sg0, sg1,
                  num_acc, den_acc, sem0, sem1, ssem):
    cid = lax.axis_index("c")
    sid = lax.axis_index("s")
    wid = sid * NC + cid
    base = wid * EPW

    icps = [pltpu.async_copy(src_h.at[pl.ds(base, EPW)], src_v, sem0)]
    for t in range(NCH):
        icps.append(pltpu.async_copy(dst_h.at[pl.ds(base + t * CH, CH)],
                                     dst_v.at[t], sem1))

    @pl.when(sid == 0)
    def _():
        pltpu.sync_copy(zeros_h, num_acc)
        pltpu.sync_copy(zeros_h, den_acc)

    slots = ((db0, e0, c0, sb0, sg0, sem0), (db1, e1, c1, sb1, sg1, sem1))

    def fire(t):
        dbr, er, cr, _, _, sm = slots[t % 2]
        return (pltpu.async_copy(dbh.at[src_v.at[pl.ds(t * CH, CH)]], dbr, sm),
                pltpu.async_copy(eh.at[dst_v.at[t]], er, sm),
                pltpu.async_copy(ce.at[pl.ds(base + t * CH, CH)], cr, sm))

    for cp in icps:
        cp.wait()
    cps = fire(0)
    plsc.subcore_barrier()

    scat = ()
    for t in range(NCH):
        nxt = fire(t + 1) if t + 1 < NCH else ()
        for cp in cps:
            cp.wait()
        dbr, er, cr, sb, sg, _ = slots[t % 2]
        for cp in scat:
            cp.wait()

        @plsc.parallel_loop(0, CH, unroll=1)
        def _(i):
            for j in range(D // L):
                sl = pl.ds(j * L, L)
                v = dbr[i, sl] + er[i, sl] + cr[i, sl]
                s = 1.0 / (1.0 + jnp.exp(-v))
                sb[i, sl] = s * dbr[i, pl.ds(D + j * L, L)]
                sg[i, sl] = s

        scat = (pltpu.async_copy(sb, num_acc.at[dst_v.at[t]], ssem, add=True),
                pltpu.async_copy(sg, den_acc.at[dst_v.at[t]], ssem, add=True))
        cps = nxt
    for cp in scat:
        cp.wait()

    plsc.subcore_barrier()

    @pl.when(sid == 0)
    def _():
        pltpu.sync_copy(num_acc, nd_o.at[cid, 0])
        pltpu.sync_copy(den_acc, nd_o.at[cid, 1])


@functools.partial(
    pl.kernel, mesh=_sc_mesh,
    out_type=jax.ShapeDtypeStruct((E, D), jnp.float32),
    scratch_types=[
        pltpu.VMEM((NCH, CH), jnp.int32),
        pltpu.VMEM((NCH, CH), jnp.int32),
        pltpu.VMEM((CH, D), jnp.float32),
        pltpu.VMEM((CH, D), jnp.float32),
        pltpu.VMEM((CH, D), jnp.float32),
        pltpu.VMEM((CH, D), jnp.float32),
        pltpu.VMEM((CH, D), jnp.float32),
        pltpu.VMEM((CH, D), jnp.float32),
        pltpu.SemaphoreType.DMA,
        pltpu.SemaphoreType.DMA,
        pltpu.SemaphoreType.DMA,
    ])
def _sc_final(shat, that, src_h, dst_h, out_o,
              src_v, dst_v, s0, t0, s1, t1, o0, o1, sem0, sem1, osem):
    cid = lax.axis_index("c")
    sid = lax.axis_index("s")
    wid = sid * NC + cid
    base = wid * EPW

    for t in range(NCH):
        pltpu.sync_copy(src_h.at[pl.ds(base + t * CH, CH)], src_v.at[t])
        pltpu.sync_copy(dst_h.at[pl.ds(base + t * CH, CH)], dst_v.at[t])

    slots = ((s0, t0, o0, sem0), (s1, t1, o1, sem1))

    def fire(t):
        sr, tr, _, sm = slots[t % 2]
        return (pltpu.async_copy(shat.at[src_v.at[t]], sr, sm),
                pltpu.async_copy(that.at[dst_v.at[t]], tr, sm))

    cps = fire(0)
    ost = ()
    for t in range(NCH):
        nxt = fire(t + 1) if t + 1 < NCH else ()
        for cp in cps:
            cp.wait()
        sr, tr, orows, _ = slots[t % 2]
        for cp in ost:
            cp.wait()

        @plsc.parallel_loop(0, CH, unroll=1)
        def _(i):
            for j in range(D // L):
                sl = pl.ds(j * L, L)
                orows[i, sl] = jnp.maximum(sr[i, sl] + tr[i, sl], 0.0)

        ost = (pltpu.async_copy(orows, out_o.at[pl.ds(base + t * CH, CH)],
                                osem),)
        cps = nxt
    for cp in ost:
        cp.wait()


def _row(x):
    return x.reshape(1, -1)


def kernel(emb_h, h, e, edge_index, params):
    src = edge_index[0]
    dst = edge_index[1]
    p1, p2 = params['gcn']
    enc = params['enc']
    zeros_nd = jnp.zeros((N, D), jnp.float32)

    b2l_row = _row(params['conv_b'][0] * jnp.sum(params['W2'], axis=0)
                   + params['b2'])
    w = params['conv_w'][0]
    band = (jnp.eye(D, k=1, dtype=jnp.float32),
            jnp.eye(D, dtype=jnp.float32),
            jnp.eye(D, k=-1, dtype=jnp.float32))
    Ms = w[0, 0] * band[0] + w[0, 1] * band[1] + w[0, 2] * band[2]
    Md = w[1, 0] * band[0] + w[1, 1] * band[1] + w[1, 2] * band[2]

    nd = jax.ShapeDtypeStruct((N, D), jnp.float32)
    nd2 = jax.ShapeDtypeStruct((N, 2 * D), jnp.float32)
    ed = jax.ShapeDtypeStruct((E, D), jnp.float32)

    dstc = dst.reshape(E, 1)
    ah1, dbh1, eh1, ce1 = pl.pallas_call(
        _tc_prep,
        out_shape=(nd, nd2, nd, ed),
    )(h, e, emb_h,
      p1['A'], _row(p1['Ab']), p1['B'], _row(p1['Bb']),
      p1['C'], _row(p1['Cb']), p1['D'], _row(p1['Db']),
      p1['E'], _row(p1['Eb']))

    ce2, shat, ldw = pl.pallas_call(
        _tc_prep_b,
        out_shape=(ed, nd, nd),
    )(e, emb_h, p2['C'], _row(p2['Cb']),
      Ms, Md, params['W2'], b2l_row, _row(params['b4']),
      _row(params['bn_g']), _row(params['bn_b']))

    nd1p = _sc_edge_pass(dbh1, eh1, ce1, src, dst, zeros_nd)

    h1, ah2, dbh2, eh2 = pl.pallas_call(
        _tc_mid,
        out_shape=(nd, nd, nd2, nd),
    )(h, ah1, nd1p, _row(p1['bnh_g']), _row(p1['bnh_b']),
      p2['A'], _row(p2['Ab']), p2['B'], _row(p2['Bb']),
      p2['D'], _row(p2['Db']), p2['E'], _row(p2['Eb']))

    nd2p = _sc_edge_pass(dbh2, eh2, ce2, src, dst, zeros_nd)

    se = pl.pallas_call(
        _tc_tail_pre,
        out_shape=ed,
    )(shat, ldw, src.reshape(E, 1), dstc, _row(params['bn_g']))

    return pl.pallas_call(
        _tc_tail,
        out_shape=ed,
    )(h1, ah2, nd2p, _row(p2['bnh_g']), _row(p2['bnh_b']),
      emb_h,
      enc['Wq'], _row(enc['bq']), enc['Wk'], _row(enc['bk']),
      enc['Wv'], _row(enc['bv']), enc['Wm'], _row(enc['bm']),
      _row(enc['ln1_g']), _row(enc['ln1_b']),
      enc['W1'], _row(enc['b1']), enc['W2'], _row(enc['b2']),
      _row(enc['ln2_g']), _row(enc['ln2_b']),
      params['W4'], _row(params['bn_g']),
      se, dstc)


# speedup vs baseline: 1.0734x; 1.0734x over previous
import functools

import jax
import jax.numpy as jnp
from jax import lax
from jax.experimental import pallas as pl
from jax.experimental.pallas import tpu as pltpu
from jax.experimental.pallas import tpu_sc as plsc

N = 256
E = 8192
D = 128
NC = 2
NS = 16
NW = NC * NS
EPW = E // NW
CH = 64
NCH = EPW // CH
L = 16


def _elu1(x):
    return jnp.where(x > 0, x + 1.0, jnp.exp(x))


def _ln(x, g, b):
    mu = x.mean(-1, keepdims=True)
    var = ((x - mu) ** 2).mean(-1, keepdims=True)
    return g * (x - mu) / jnp.sqrt(var + 1e-5) + b


def _tc_prep(h, e, emb_h, A1, Ab1, B1, Bb1, C1, Cb1, D1, Db1, E1, Eb1,
             ah_o, dbh_o, eh_o, ce1_o):
    hh = h[...]
    ah_o[...] = hh @ A1[...] + Ab1[...]
    dbh_o[:, :D] = hh @ D1[...] + Db1[...]
    dbh_o[:, D:] = hh @ B1[...] + Bb1[...]
    eh_o[...] = hh @ E1[...] + Eb1[...]
    ce1_o[...] = e[...] @ C1[...] + Cb1[...]


def _tc_prep_b(e, emb_h, C2, Cb2, Ms, Md, W2l, b2l, b4, bng, bnb,
               ce2_o, shat_o, ldw_o):
    ce2_o[...] = e[...] @ C2[...] + Cb2[...]
    emb = emb_h[...]
    Ls = (emb @ Ms[...]) @ W2l[...]
    LdW = (emb @ Md[...]) @ W2l[...]
    gam = bng[...] / jnp.sqrt(1.0 + 1e-5)
    shat_o[...] = gam * (Ls + b2l[...] + b4[...]) + bnb[...]
    ldw_o[...] = LdW


def _tc_mid(h, ah1, ndp, g1, b1_, A2, Ab2, B2, Bb2, D2, Db2,
            E2, Eb2,
            h1_o, ah_o, dbh_o, eh_o):
    num = ndp[0, 0] + ndp[1, 0]
    den = ndp[0, 1] + ndp[1, 1]
    h_new = ah1[...] + num / (den + 1e-6)
    hb = g1[...] * h_new / jnp.sqrt(1.0 + 1e-5) + b1_[...]
    h1 = h[...] + jnp.maximum(hb, 0.0)
    h1_o[...] = h1
    ah_o[...] = h1 @ A2[...] + Ab2[...]
    dbh_o[:, :D] = h1 @ D2[...] + Db2[...]
    dbh_o[:, D:] = h1 @ B2[...] + Bb2[...]
    eh_o[...] = h1 @ E2[...] + Eb2[...]


def _tc_tail_pre(shat, ldw, srcc, dstc, bng, se_o):
    gam = bng[...] / jnp.sqrt(1.0 + 1e-5)
    col = lax.broadcasted_iota(jnp.int32, (E, N), 1)
    oh_src = (srcc[...] == col).astype(jnp.float32)
    oh_dst = (dstc[...] == col).astype(jnp.float32)
    se_o[...] = (jnp.dot(oh_src, shat[...], preferred_element_type=jnp.float32)
                 + jnp.dot(oh_dst, gam * ldw[...],
                           preferred_element_type=jnp.float32))


def _tc_tail(h1, ah2, ndp, g2, b2_, emb_h,
             Wq, bq, Wk, bk, Wv, bv, Wm, bm, ln1g, ln1b,
             W1, b1e, W2e, b2e, ln2g, ln2b, W4, bng,
             se, dstc,
             lre_o):
    num = ndp[0, 0] + ndp[1, 0]
    den = ndp[0, 1] + ndp[1, 1]
    h_new = ah2[...] + num / (den + 1e-6)
    hb = g2[...] * h_new / jnp.sqrt(1.0 + 1e-5) + b2_[...]
    h2 = h1[...] + jnp.maximum(hb, 0.0)
    g = h2.mean(0, keepdims=True)

    emb = emb_h[...]
    Q = _elu1(emb @ Wq[...] + bq[...])
    krow = _elu1(g @ Wk[...] + bk[...])
    vrow = g @ Wv[...] + bv[...]
    s = (Q * krow).sum(-1, keepdims=True)
    ns = jnp.float32(N) * s
    msg = (ns / (ns + 1e-6)) * vrow
    msg = _ln(msg @ Wm[...] + bm[...], ln1g[...], ln1b[...])
    y = jnp.concatenate([emb, msg], axis=-1)
    y = jnp.maximum(y @ W1[...] + b1e[...], 0.0) @ W2e[...] + b2e[...]
    qfea = emb + _ln(y, ln2g[...], ln2b[...])

    Q2 = _elu1(qfea @ Wq[...] + bq[...])
    K2 = _elu1(qfea @ Wk[...] + bk[...])
    V2 = qfea @ Wv[...] + bv[...]
    KV = jnp.float32(N) * lax.dot_general(
        K2, V2, (((0,), (0,)), ((), ())), preferred_element_type=jnp.float32)
    ksum = jnp.float32(N) * K2.sum(0, keepdims=True)
    Z = (Q2 * ksum).sum(-1, keepdims=True)
    msg2 = (Q2 @ KV) / (Z + 1e-6)
    msg2 = _ln(msg2 @ Wm[...] + bm[...], ln1g[...], ln1b[...])
    y2 = jnp.concatenate([qfea, msg2], axis=-1)
    y2 = jnp.maximum(y2 @ W1[...] + b1e[...], 0.0) @ W2e[...] + b2e[...]
    arm = qfea + _ln(y2, ln2g[...], ln2b[...])

    gam = bng[...] / jnp.sqrt(1.0 + 1e-5)
    G = gam * (arm @ W4[...])

    col = lax.broadcasted_iota(jnp.int32, (E, N), 1)
    oh_dst = (dstc[...] == col).astype(jnp.float32)
    lre_o[...] = jnp.maximum(
        se[...] + jnp.dot(oh_dst, G, preferred_element_type=jnp.float32),
        0.0)


_sc_mesh = plsc.VectorSubcoreMesh(core_axis_name="c", subcore_axis_name="s")


@functools.partial(
    pl.kernel, mesh=_sc_mesh,
    out_type=jax.ShapeDtypeStruct((NC, 2, N, D), jnp.float32),
    scratch_types=[
        pltpu.VMEM((EPW,), jnp.int32),
        pltpu.VMEM((NCH, CH), jnp.int32),
        pltpu.VMEM((CH, 2 * D), jnp.float32),
        pltpu.VMEM((CH, 2 * D), jnp.float32),
        pltpu.VMEM((CH, D), jnp.float32),
        pltpu.VMEM((CH, D), jnp.float32),
        pltpu.VMEM((CH, D), jnp.float32),
        pltpu.VMEM((CH, D), jnp.float32),
        pltpu.VMEM((CH, D), jnp.float32),
        pltpu.VMEM((CH, D), jnp.float32),
        pltpu.VMEM((CH, D), jnp.float32),
        pltpu.VMEM((CH, D), jnp.float32),
        pltpu.VMEM_SHARED((N, D), jnp.float32),
        pltpu.VMEM_SHARED((N, D), jnp.float32),
        pltpu.SemaphoreType.DMA,
        pltpu.SemaphoreType.DMA,
        pltpu.SemaphoreType.DMA,
    ])
def _sc_edge_pass(dbh, eh, ce, src_h, dst_h, zeros_h,
                  nd_o,
                  src_v, dst_v, db0, db1, e0, e1, c0, c1,
                  sb0, sb1, sg0, sg1,
                  num_acc, den_acc, sem0, sem1, ssem):
    cid = lax.axis_index("c")
    sid = lax.axis_index("s")
    wid = sid * NC + cid
    base = wid * EPW

    icps = [pltpu.async_copy(src_h.at[pl.ds(base, EPW)], src_v, sem0)]
    for t in range(NCH):
        icps.append(pltpu.async_copy(dst_h.at[pl.ds(base + t * CH, CH)],
                                     dst_v.at[t], sem1))

    @pl.when(sid == 0)
    def _():
        pltpu.sync_copy(zeros_h, num_acc)
        pltpu.sync_copy(zeros_h, den_acc)

    slots = ((db0, e0, c0, sb0, sg0, sem0), (db1, e1, c1, sb1, sg1, sem1))

    def fire(t):
        dbr, er, cr, _, _, sm = slots[t % 2]
        return (pltpu.async_copy(dbh.at[src_v.at[pl.ds(t * CH, CH)]], dbr, sm),
                pltpu.async_copy(eh.at[dst_v.at[t]], er, sm),
                pltpu.async_copy(ce.at[pl.ds(base + t * CH, CH)], cr, sm))

    for cp in icps:
        cp.wait()
    cps = fire(0)
    plsc.subcore_barrier()

    scat = ()
    for t in range(NCH):
        nxt = fire(t + 1) if t + 1 < NCH else ()
        for cp in cps:
            cp.wait()
        dbr, er, cr, sb, sg, _ = slots[t % 2]
        for cp in scat:
            cp.wait()

        @plsc.parallel_loop(0, CH, unroll=2)
        def _(i):
            for j in range(D // L):
                sl = pl.ds(j * L, L)
                v = dbr[i, sl] + er[i, sl] + cr[i, sl]
                s = 1.0 / (1.0 + jnp.exp(-v))
                sb[i, sl] = s * dbr[i, pl.ds(D + j * L, L)]
                sg[i, sl] = s

        scat = (pltpu.async_copy(sb, num_acc.at[dst_v.at[t]], ssem, add=True),
                pltpu.async_copy(sg, den_acc.at[dst_v.at[t]], ssem, add=True))
        cps = nxt
    for cp in scat:
        cp.wait()

    plsc.subcore_barrier()

    @pl.when(sid == 0)
    def _():
        pltpu.sync_copy(num_acc, nd_o.at[cid, 0])
        pltpu.sync_copy(den_acc, nd_o.at[cid, 1])


@functools.partial(
    pl.kernel, mesh=_sc_mesh,
    out_type=jax.ShapeDtypeStruct((E, D), jnp.float32),
    scratch_types=[
        pltpu.VMEM((NCH, CH), jnp.int32),
        pltpu.VMEM((NCH, CH), jnp.int32),
        pltpu.VMEM((CH, D), jnp.float32),
        pltpu.VMEM((CH, D), jnp.float32),
        pltpu.VMEM((CH, D), jnp.float32),
        pltpu.VMEM((CH, D), jnp.float32),
        pltpu.VMEM((CH, D), jnp.float32),
        pltpu.VMEM((CH, D), jnp.float32),
        pltpu.SemaphoreType.DMA,
        pltpu.SemaphoreType.DMA,
        pltpu.SemaphoreType.DMA,
    ])
def _sc_final(shat, that, src_h, dst_h, out_o,
              src_v, dst_v, s0, t0, s1, t1, o0, o1, sem0, sem1, osem):
    cid = lax.axis_index("c")
    sid = lax.axis_index("s")
    wid = sid * NC + cid
    base = wid * EPW

    for t in range(NCH):
        pltpu.sync_copy(src_h.at[pl.ds(base + t * CH, CH)], src_v.at[t])
        pltpu.sync_copy(dst_h.at[pl.ds(base + t * CH, CH)], dst_v.at[t])

    slots = ((s0, t0, o0, sem0), (s1, t1, o1, sem1))

    def fire(t):
        sr, tr, _, sm = slots[t % 2]
        return (pltpu.async_copy(shat.at[src_v.at[t]], sr, sm),
                pltpu.async_copy(that.at[dst_v.at[t]], tr, sm))

    cps = fire(0)
    ost = ()
    for t in range(NCH):
        nxt = fire(t + 1) if t + 1 < NCH else ()
        for cp in cps:
            cp.wait()
        sr, tr, orows, _ = slots[t % 2]
        for cp in ost:
            cp.wait()

        @plsc.parallel_loop(0, CH, unroll=2)
        def _(i):
            for j in range(D // L):
                sl = pl.ds(j * L, L)
                orows[i, sl] = jnp.maximum(sr[i, sl] + tr[i, sl], 0.0)

        ost = (pltpu.async_copy(orows, out_o.at[pl.ds(base + t * CH, CH)],
                                osem),)
        cps = nxt
    for cp in ost:
        cp.wait()


def _row(x):
    return x.reshape(1, -1)


def kernel(emb_h, h, e, edge_index, params):
    src = edge_index[0]
    dst = edge_index[1]
    p1, p2 = params['gcn']
    enc = params['enc']
    zeros_nd = jnp.zeros((N, D), jnp.float32)

    b2l_row = _row(params['conv_b'][0] * jnp.sum(params['W2'], axis=0)
                   + params['b2'])
    w = params['conv_w'][0]
    band = (jnp.eye(D, k=1, dtype=jnp.float32),
            jnp.eye(D, dtype=jnp.float32),
            jnp.eye(D, k=-1, dtype=jnp.float32))
    Ms = w[0, 0] * band[0] + w[0, 1] * band[1] + w[0, 2] * band[2]
    Md = w[1, 0] * band[0] + w[1, 1] * band[1] + w[1, 2] * band[2]

    nd = jax.ShapeDtypeStruct((N, D), jnp.float32)
    nd2 = jax.ShapeDtypeStruct((N, 2 * D), jnp.float32)
    ed = jax.ShapeDtypeStruct((E, D), jnp.float32)

    dstc = dst.reshape(E, 1)
    ah1, dbh1, eh1, ce1 = pl.pallas_call(
        _tc_prep,
        out_shape=(nd, nd2, nd, ed),
    )(h, e, emb_h,
      p1['A'], _row(p1['Ab']), p1['B'], _row(p1['Bb']),
      p1['C'], _row(p1['Cb']), p1['D'], _row(p1['Db']),
      p1['E'], _row(p1['Eb']))

    ce2, shat, ldw = pl.pallas_call(
        _tc_prep_b,
        out_shape=(ed, nd, nd),
    )(e, emb_h, p2['C'], _row(p2['Cb']),
      Ms, Md, params['W2'], b2l_row, _row(params['b4']),
      _row(params['bn_g']), _row(params['bn_b']))

    nd1p = _sc_edge_pass(dbh1, eh1, ce1, src, dst, zeros_nd)

    h1, ah2, dbh2, eh2 = pl.pallas_call(
        _tc_mid,
        out_shape=(nd, nd, nd2, nd),
    )(h, ah1, nd1p, _row(p1['bnh_g']), _row(p1['bnh_b']),
      p2['A'], _row(p2['Ab']), p2['B'], _row(p2['Bb']),
      p2['D'], _row(p2['Db']), p2['E'], _row(p2['Eb']))

    nd2p = _sc_edge_pass(dbh2, eh2, ce2, src, dst, zeros_nd)

    se = pl.pallas_call(
        _tc_tail_pre,
        out_shape=ed,
    )(shat, ldw, src.reshape(E, 1), dstc, _row(params['bn_g']))

    return pl.pallas_call(
        _tc_tail,
        out_shape=ed,
    )(h1, ah2, nd2p, _row(p2['bnh_g']), _row(p2['bnh_b']),
      emb_h,
      enc['Wq'], _row(enc['bq']), enc['Wk'], _row(enc['bk']),
      enc['Wv'], _row(enc['bv']), enc['Wm'], _row(enc['bm']),
      _row(enc['ln1_g']), _row(enc['ln1_b']),
      enc['W1'], _row(enc['b1']), enc['W2'], _row(enc['b2']),
      _row(enc['ln2_g']), _row(enc['ln2_b']),
      params['W4'], _row(params['bn_g']),
      se, dstc)


# speedup vs baseline: 1.0760x; 1.0024x over previous
import functools

import jax
import jax.numpy as jnp
from jax import lax
from jax.experimental import pallas as pl
from jax.experimental.pallas import tpu as pltpu
from jax.experimental.pallas import tpu_sc as plsc

N = 256
E = 8192
D = 128
NC = 2
NS = 16
NW = NC * NS
EPW = E // NW
CH = 64
NCH = EPW // CH
L = 16


def _elu1(x):
    return jnp.where(x > 0, x + 1.0, jnp.exp(x))


def _ln(x, g, b):
    mu = x.mean(-1, keepdims=True)
    var = ((x - mu) ** 2).mean(-1, keepdims=True)
    return g * (x - mu) / jnp.sqrt(var + 1e-5) + b


def _tc_prep(h, e, emb_h, A1, Ab1, B1, Bb1, C1, Cb1, D1, Db1, E1, Eb1,
             ah_o, dbh_o, eh_o, ce1_o):
    hh = h[...]
    ah_o[...] = hh @ A1[...] + Ab1[...]
    dbh_o[:, :D] = hh @ D1[...] + Db1[...]
    dbh_o[:, D:] = hh @ B1[...] + Bb1[...]
    eh_o[...] = hh @ E1[...] + Eb1[...]
    ce1_o[...] = e[...] @ C1[...] + Cb1[...]


def _tc_prep_b(e, emb_h, C2, Cb2, Ms, Md, W2l, b2l, b4, bng, bnb,
               ce2_o, shat_o, ldw_o):
    ce2_o[...] = e[...] @ C2[...] + Cb2[...]
    emb = emb_h[...]
    Ls = (emb @ Ms[...]) @ W2l[...]
    LdW = (emb @ Md[...]) @ W2l[...]
    gam = bng[...] / jnp.sqrt(1.0 + 1e-5)
    shat_o[...] = gam * (Ls + b2l[...] + b4[...]) + bnb[...]
    ldw_o[...] = LdW


def _tc_mid(h, ah1, ndp, g1, b1_, A2, Ab2, B2, Bb2, D2, Db2,
            E2, Eb2,
            h1_o, ah_o, dbh_o, eh_o):
    num = ndp[0, 0] + ndp[1, 0]
    den = ndp[0, 1] + ndp[1, 1]
    h_new = ah1[...] + num / (den + 1e-6)
    hb = g1[...] * h_new / jnp.sqrt(1.0 + 1e-5) + b1_[...]
    h1 = h[...] + jnp.maximum(hb, 0.0)
    h1_o[...] = h1
    ah_o[...] = h1 @ A2[...] + Ab2[...]
    dbh_o[:, :D] = h1 @ D2[...] + Db2[...]
    dbh_o[:, D:] = h1 @ B2[...] + Bb2[...]
    eh_o[...] = h1 @ E2[...] + Eb2[...]


def _tc_tail_pre(shat, ldw, srcc, dstc, bng, se_o):
    gam = bng[...] / jnp.sqrt(1.0 + 1e-5)
    col = lax.broadcasted_iota(jnp.int32, (E, N), 1)
    oh_src = (srcc[...] == col).astype(jnp.float32)
    oh_dst = (dstc[...] == col).astype(jnp.float32)
    se_o[...] = (jnp.dot(oh_src, shat[...], preferred_element_type=jnp.float32)
                 + jnp.dot(oh_dst, gam * ldw[...],
                           preferred_element_type=jnp.float32))


def _tc_tail(h1, ah2, ndp, g2, b2_, emb_h,
             Wq, bq, Wk, bk, Wv, bv, Wm, bm, ln1g, ln1b,
             W1, b1e, W2e, b2e, ln2g, ln2b, W4, bng,
             se, dstc,
             lre_o):
    num = ndp[0, 0] + ndp[1, 0]
    den = ndp[0, 1] + ndp[1, 1]
    h_new = ah2[...] + num / (den + 1e-6)
    hb = g2[...] * h_new / jnp.sqrt(1.0 + 1e-5) + b2_[...]
    h2 = h1[...] + jnp.maximum(hb, 0.0)
    g = h2.mean(0, keepdims=True)

    emb = emb_h[...]
    Q = _elu1(emb @ Wq[...] + bq[...])
    krow = _elu1(g @ Wk[...] + bk[...])
    vrow = g @ Wv[...] + bv[...]
    s = (Q * krow).sum(-1, keepdims=True)
    ns = jnp.float32(N) * s
    msg = (ns / (ns + 1e-6)) * vrow
    msg = _ln(msg @ Wm[...] + bm[...], ln1g[...], ln1b[...])
    y = jnp.concatenate([emb, msg], axis=-1)
    y = jnp.maximum(y @ W1[...] + b1e[...], 0.0) @ W2e[...] + b2e[...]
    qfea = emb + _ln(y, ln2g[...], ln2b[...])

    Q2 = _elu1(qfea @ Wq[...] + bq[...])
    K2 = _elu1(qfea @ Wk[...] + bk[...])
    V2 = qfea @ Wv[...] + bv[...]
    KV = jnp.float32(N) * lax.dot_general(
        K2, V2, (((0,), (0,)), ((), ())), preferred_element_type=jnp.float32)
    ksum = jnp.float32(N) * K2.sum(0, keepdims=True)
    Z = (Q2 * ksum).sum(-1, keepdims=True)
    msg2 = (Q2 @ KV) / (Z + 1e-6)
    msg2 = _ln(msg2 @ Wm[...] + bm[...], ln1g[...], ln1b[...])
    y2 = jnp.concatenate([qfea, msg2], axis=-1)
    y2 = jnp.maximum(y2 @ W1[...] + b1e[...], 0.0) @ W2e[...] + b2e[...]
    arm = qfea + _ln(y2, ln2g[...], ln2b[...])

    gam = bng[...] / jnp.sqrt(1.0 + 1e-5)
    G = gam * (arm @ W4[...])

    col = lax.broadcasted_iota(jnp.int32, (E, N), 1)
    oh_dst = (dstc[...] == col).astype(jnp.float32)
    lre_o[...] = jnp.maximum(
        se[...] + jnp.dot(oh_dst, G, preferred_element_type=jnp.float32),
        0.0)


_sc_mesh = plsc.VectorSubcoreMesh(core_axis_name="c", subcore_axis_name="s")


@functools.partial(
    pl.kernel, mesh=_sc_mesh,
    out_type=jax.ShapeDtypeStruct((NC, 2, N, D), jnp.float32),
    scratch_types=[
        pltpu.VMEM((EPW,), jnp.int32),
        pltpu.VMEM((NCH, CH), jnp.int32),
        pltpu.VMEM((CH, 2 * D), jnp.float32),
        pltpu.VMEM((CH, 2 * D), jnp.float32),
        pltpu.VMEM((CH, D), jnp.float32),
        pltpu.VMEM((CH, D), jnp.float32),
        pltpu.VMEM((CH, D), jnp.float32),
        pltpu.VMEM((CH, D), jnp.float32),
        pltpu.VMEM((CH, D), jnp.float32),
        pltpu.VMEM((CH, D), jnp.float32),
        pltpu.VMEM((CH, D), jnp.float32),
        pltpu.VMEM((CH, D), jnp.float32),
        pltpu.VMEM_SHARED((N, D), jnp.float32),
        pltpu.VMEM_SHARED((N, D), jnp.float32),
        pltpu.SemaphoreType.DMA,
        pltpu.SemaphoreType.DMA,
        pltpu.SemaphoreType.DMA,
    ])
def _sc_edge_pass(dbh, eh, ce, src_h, dst_h, zeros_h,
                  nd_o,
                  src_v, dst_v, db0, db1, e0, e1, c0, c1,
                  sb0, sb1, sg0, sg1,
                  num_acc, den_acc, sem0, sem1, ssem):
    cid = lax.axis_index("c")
    sid = lax.axis_index("s")
    wid = sid * NC + cid
    base = wid * EPW

    icps = [pltpu.async_copy(src_h.at[pl.ds(base, EPW)], src_v, sem0)]
    for t in range(NCH):
        icps.append(pltpu.async_copy(dst_h.at[pl.ds(base + t * CH, CH)],
                                     dst_v.at[t], sem1))

    @pl.when(sid == 0)
    def _():
        pltpu.sync_copy(zeros_h, num_acc)
        pltpu.sync_copy(zeros_h, den_acc)

    slots = ((db0, e0, c0, sb0, sg0, sem0), (db1, e1, c1, sb1, sg1, sem1))

    def fire(t):
        dbr, er, cr, _, _, sm = slots[t % 2]
        return (pltpu.async_copy(dbh.at[src_v.at[pl.ds(t * CH, CH)]], dbr, sm),
                pltpu.async_copy(eh.at[dst_v.at[t]], er, sm),
                pltpu.async_copy(ce.at[pl.ds(base + t * CH, CH)], cr, sm))

    for cp in icps:
        cp.wait()
    cps = fire(0)
    plsc.subcore_barrier()

    scat = ()
    for t in range(NCH):
        nxt = fire(t + 1) if t + 1 < NCH else ()
        for cp in cps:
            cp.wait()
        dbr, er, cr, sb, sg, _ = slots[t % 2]
        for cp in scat:
            cp.wait()

        @plsc.parallel_loop(0, CH, unroll=2)
        def _(i):
            for j in range(D // L):
                sl = pl.ds(j * L, L)
                v = dbr[i, sl] + er[i, sl] + cr[i, sl]
                s = 1.0 / (1.0 + jnp.exp(-v))
                sb[i, sl] = s * dbr[i, pl.ds(D + j * L, L)]
                sg[i, sl] = s

        scat = (pltpu.async_copy(sb, num_acc.at[dst_v.at[t]], ssem, add=True),
                pltpu.async_copy(sg, den_acc.at[dst_v.at[t]], ssem, add=True))
        cps = nxt
    for cp in scat:
        cp.wait()

    plsc.subcore_barrier()

    @pl.when(sid == 0)
    def _():
        pltpu.sync_copy(num_acc, nd_o.at[cid, 0])
        pltpu.sync_copy(den_acc, nd_o.at[cid, 1])


def _row(x):
    return x.reshape(1, -1)


def kernel(emb_h, h, e, edge_index, params):
    src = edge_index[0]
    dst = edge_index[1]
    p1, p2 = params['gcn']
    enc = params['enc']
    zeros_nd = jnp.zeros((N, D), jnp.float32)

    b2l_row = _row(params['conv_b'][0] * jnp.sum(params['W2'], axis=0)
                   + params['b2'])
    w = params['conv_w'][0]
    band = (jnp.eye(D, k=1, dtype=jnp.float32),
            jnp.eye(D, dtype=jnp.float32),
            jnp.eye(D, k=-1, dtype=jnp.float32))
    Ms = w[0, 0] * band[0] + w[0, 1] * band[1] + w[0, 2] * band[2]
    Md = w[1, 0] * band[0] + w[1, 1] * band[1] + w[1, 2] * band[2]

    nd = jax.ShapeDtypeStruct((N, D), jnp.float32)
    nd2 = jax.ShapeDtypeStruct((N, 2 * D), jnp.float32)
    ed = jax.ShapeDtypeStruct((E, D), jnp.float32)

    dstc = dst.reshape(E, 1)
    ah1, dbh1, eh1, ce1 = pl.pallas_call(
        _tc_prep,
        out_shape=(nd, nd2, nd, ed),
    )(h, e, emb_h,
      p1['A'], _row(p1['Ab']), p1['B'], _row(p1['Bb']),
      p1['C'], _row(p1['Cb']), p1['D'], _row(p1['Db']),
      p1['E'], _row(p1['Eb']))

    ce2, shat, ldw = pl.pallas_call(
        _tc_prep_b,
        out_shape=(ed, nd, nd),
    )(e, emb_h, p2['C'], _row(p2['Cb']),
      Ms, Md, params['W2'], b2l_row, _row(params['b4']),
      _row(params['bn_g']), _row(params['bn_b']))

    nd1p = _sc_edge_pass(dbh1, eh1, ce1, src, dst, zeros_nd)

    h1, ah2, dbh2, eh2 = pl.pallas_call(
        _tc_mid,
        out_shape=(nd, nd, nd2, nd),
    )(h, ah1, nd1p, _row(p1['bnh_g']), _row(p1['bnh_b']),
      p2['A'], _row(p2['Ab']), p2['B'], _row(p2['Bb']),
      p2['D'], _row(p2['Db']), p2['E'], _row(p2['Eb']))

    nd2p = _sc_edge_pass(dbh2, eh2, ce2, src, dst, zeros_nd)

    se = pl.pallas_call(
        _tc_tail_pre,
        out_shape=ed,
    )(shat, ldw, src.reshape(E, 1), dstc, _row(params['bn_g']))

    return pl.pallas_call(
        _tc_tail,
        out_shape=ed,
    )(h1, ah2, nd2p, _row(p2['bnh_g']), _row(p2['bnh_b']),
      emb_h,
      enc['Wq'], _row(enc['bq']), enc['Wk'], _row(enc['bk']),
      enc['Wv'], _row(enc['bv']), enc['Wm'], _row(enc['bm']),
      _row(enc['ln1_g']), _row(enc['ln1_b']),
      enc['W1'], _row(enc['b1']), enc['W2'], _row(enc['b2']),
      _row(enc['ln2_g']), _row(enc['ln2_b']),
      params['W4'], _row(params['bn_g']),
      se, dstc)
